# baseline (speedup 1.0000x reference)
import functools

import jax
import jax.numpy as jnp
from jax import lax
from jax.experimental import pallas as pl
from jax.experimental.pallas import tpu as pltpu
from jax.experimental.pallas import tpu_sc as plsc

B, S, D = 1024, 200, 64
VOCAB = 1000000
VOCAB2 = VOCAB // 2
NC, NS = 2, 16
NW = NC * NS
LANES = 16
BBLK = 128
NBLK = B // BBLK
NBLOCKS = S * NBLK
BLK_PER_W = NBLOCKS // NW
PCOLS = 256

SLAB = 256
NSLAB = VOCAB // SLAB
TAIL0 = NSLAB * SLAB
TAIL = VOCAB - TAIL0
SLAB_PER_W = NSLAB // NW
SLAB_REM = NSLAB - SLAB_PER_W * NW

_SC_PARAMS = pltpu.CompilerParams(
    use_tc_tiling_on_sc=True, needs_layout_passes=False
)


def _pack_body(tokT_hbm, tail_hbm, packed_hbm, in_v, out_v, sem):
    wid = lax.axis_index("s") * NC + lax.axis_index("c")
    n_w = SLAB_PER_W + jnp.where(wid < SLAB_REM, 1, 0)
    k0 = SLAB_PER_W * wid + jnp.minimum(wid, SLAB_REM)

    iota = jax.lax.iota(jnp.int32, LANES)
    fvecs = [iota + (g * LANES) for g in range(D // LANES)]

    def transpose_rows(nrows):
        def j_body(j, c):
            ce = jnp.full((LANES,), 2 * j, jnp.int32)
            co = jnp.full((LANES,), 2 * j + 1, jnp.int32)
            for g in range(D // LANES):
                vals = plsc.load_gather(in_v, [fvecs[g], ce])
                out_v[j, pl.ds(g * LANES, LANES)] = vals
            for g in range(D // LANES):
                vals = plsc.load_gather(in_v, [fvecs[g], co])
                out_v[j, pl.ds(D + g * LANES, LANES)] = vals
            return c

        lax.fori_loop(0, nrows, j_body, 0)

    def slab_body(k, carry):
        c0 = pl.multiple_of((k0 + k) * SLAB, SLAB)
        r0 = pl.multiple_of((k0 + k) * (SLAB // 2), SLAB // 2)
        pltpu.sync_copy(tokT_hbm.at[:, pl.ds(c0, SLAB)], in_v)
        transpose_rows(SLAB // 2)
        pltpu.sync_copy(out_v, packed_hbm.at[pl.ds(r0, SLAB // 2)])
        return carry

    lax.fori_loop(0, n_w, slab_body, 0)

    @pl.when(wid == NW - 1)
    def _tail():
        pltpu.sync_copy(tail_hbm, out_v.at[pl.ds(0, TAIL // 2)])
        pltpu.sync_copy(out_v.at[pl.ds(0, TAIL // 2)],
                        packed_hbm.at[pl.ds(TAIL0 // 2, TAIL // 2)])


@functools.partial(
    pl.kernel,
    out_type=jax.ShapeDtypeStruct((VOCAB2, 128), jnp.float32),
    mesh=plsc.VectorSubcoreMesh(core_axis_name="c", subcore_axis_name="s"),
    scratch_types=[
        pltpu.VMEM((D, SLAB), jnp.float32),
        pltpu.VMEM((SLAB // 2, 128), jnp.float32),
        pltpu.SemaphoreType.DMA,
    ],
    compiler_params=_SC_PARAMS,
)
def _pack(tokT_hbm, tail_hbm, packed_hbm, *scratch):
    _pack_body(tokT_hbm, tail_hbm, packed_hbm, *scratch)


def _emb_body(x_hbm, tok_hbm, pos_hbm, out_hbm,
              idx_v, j_v, cb_v, rows_v, ob_v, pos_v, sem):
    wid = lax.axis_index("s") * NC + lax.axis_index("c")
    t0 = wid * BLK_PER_W

    pltpu.sync_copy(pos_hbm.at[:, pl.ds(0, PCOLS)], pos_v)

    def block_body(bi, carry):
        t = t0 + bi
        s = t // NBLK
        b0 = (t % NBLK) * BBLK

        pltpu.sync_copy(x_hbm.at[s, pl.ds(b0, BBLK)], idx_v)

        for g in range(BBLK // LANES):
            sl = pl.ds(g * LANES, LANES)
            v = idx_v[sl]
            j_v[sl] = lax.shift_right_logical(v, 1)
            cb_v[sl] = lax.shift_left(lax.bitwise_and(v, 1), 6)

        pltpu.async_copy(tok_hbm.at[j_v], rows_v, sem).wait()

        iota = jax.lax.iota(jnp.int32, LANES)
        rowids = [iota + (g * LANES) for g in range(BBLK // LANES)]
        cbs = [cb_v[pl.ds(g * LANES, LANES)] for g in range(BBLK // LANES)]
        sv = jnp.full((LANES,), s, jnp.int32)

        def d_body(d, carry):
            rids, cbl = carry
            dv = jnp.full((LANES,), d, jnp.int32)
            pos_vec = plsc.load_gather(pos_v, [dv, sv])
            for g in range(BBLK // LANES):
                vals = plsc.load_gather(rows_v, [rids[g], cbl[g] + d])
                ob_v[d, pl.ds(g * LANES, LANES)] = vals + pos_vec
            return carry

        lax.fori_loop(0, D, d_body, (tuple(rowids), tuple(cbs)))

        pltpu.sync_copy(ob_v, out_hbm.at[s, :, pl.ds(b0, BBLK)])
        return carry

    lax.fori_loop(0, BLK_PER_W, block_body, 0)


@functools.partial(
    pl.kernel,
    out_type=jax.ShapeDtypeStruct((S, D, B), jnp.float32),
    mesh=plsc.VectorSubcoreMesh(core_axis_name="c", subcore_axis_name="s"),
    scratch_types=[
        pltpu.VMEM((BBLK,), jnp.int32),
        pltpu.VMEM((BBLK,), jnp.int32),
        pltpu.VMEM((BBLK,), jnp.int32),
        pltpu.VMEM((BBLK, 128), jnp.float32),
        pltpu.VMEM((D, BBLK), jnp.float32),
        pltpu.VMEM((D, PCOLS), jnp.float32),
        pltpu.SemaphoreType.DMA,
    ],
    compiler_params=_SC_PARAMS,
)
def _emb(x_hbm, tok_hbm, pos_hbm, out_hbm, *scratch):
    _emb_body(x_hbm, tok_hbm, pos_hbm, out_hbm, *scratch)


@jax.jit
def kernel(x, token_table, pos_table):
    xT = x.astype(jnp.int32).T
    tokT = token_table.T
    posT = pos_table.T
    tail = token_table[TAIL0:].reshape(TAIL // 2, 128)
    packed = _pack(tokT, tail)
    out_t = _emb(xT, packed, posT)
    return out_t.transpose(2, 0, 1)


# speedup vs baseline: 2.6465x; 2.6465x over previous
import functools

import jax
import jax.numpy as jnp
from jax import lax
from jax.experimental import pallas as pl
from jax.experimental.pallas import tpu as pltpu
from jax.experimental.pallas import tpu_sc as plsc

B, S, D = 1024, 200, 64
NC, NS = 2, 16
NW = NC * NS
ROWS_PER_W = B // NW
LANES = 16
CHUNKS = ((0, 128), (128, 72))


def _emb_body(x_hbm, tok_hbm, pos_hbm, out_hbm, idx_v, rows_v, pos_v, sem):
    wid = lax.axis_index("s") * NC + lax.axis_index("c")
    base = wid * ROWS_PER_W

    pltpu.sync_copy(pos_hbm.at[pl.ds(0, S)], pos_v)

    def row_body(r, carry):
        b = base + r
        pltpu.sync_copy(x_hbm.at[b], idx_v)
        cps = [
            pltpu.async_copy(
                tok_hbm.at[idx_v.at[pl.ds(off, n)]],
                rows_v.at[pl.ds(off, n)],
                sem,
            )
            for off, n in CHUNKS
        ]
        for cp in cps:
            cp.wait()

        def add_body(i, c):
            for d in range(D // LANES):
                sl = pl.ds(d * LANES, LANES)
                plsc.addupdate(rows_v.at[i, sl], pos_v[i, sl])
            return c

        lax.fori_loop(0, S, add_body, 0, unroll=2)

        pltpu.sync_copy(rows_v, out_hbm.at[b])
        return carry

    lax.fori_loop(0, ROWS_PER_W, row_body, 0)


@functools.partial(
    pl.kernel,
    out_type=jax.ShapeDtypeStruct((B, S, D), jnp.float32),
    mesh=plsc.VectorSubcoreMesh(core_axis_name="c", subcore_axis_name="s"),
    scratch_types=[
        pltpu.VMEM((S,), jnp.int32),
        pltpu.VMEM((S, D), jnp.float32),
        pltpu.VMEM((S, D), jnp.float32),
        pltpu.SemaphoreType.DMA,
    ],
    compiler_params=pltpu.CompilerParams(use_tc_tiling_on_sc=False),
)
def _emb(x_hbm, tok_hbm, pos_hbm, out_hbm, idx_v, rows_v, pos_v, sem):
    _emb_body(x_hbm, tok_hbm, pos_hbm, out_hbm, idx_v, rows_v, pos_v, sem)


@jax.jit
def kernel(x, token_table, pos_table):
    return _emb(x.astype(jnp.int32), token_table, pos_table)
